# baseline (speedup 1.0000x reference)
import functools

import jax
import jax.numpy as jnp
from jax import lax
from jax.experimental import pallas as pl
from jax.experimental.pallas import tpu as pltpu
from jax.experimental.pallas import tpu_sc as plsc

_R, _C = 64, 32768
_N = _R * _C
_NC, _NS, _L = 2, 16, 16
_NW = _NC * _NS
_PER_W = _N // _NW
_WORDS_W = _PER_W // 4

_mesh = plsc.VectorSubcoreMesh(core_axis_name="c", subcore_axis_name="s")


@functools.partial(
    pl.kernel,
    out_type=jax.ShapeDtypeStruct((_N,), jnp.float32),
    mesh=_mesh,
    scratch_types=[
        pltpu.VMEM((_PER_W,), jnp.float32),
        pltpu.VMEM((_PER_W,), jnp.int32),
    ],
)
def _domain_mask_sc(w_hbm, m_hbm, out_hbm, w_v, m_v):
    wid = lax.axis_index("s") * _NC + lax.axis_index("c")
    base = wid * _PER_W

    pltpu.sync_copy(w_hbm.at[pl.ds(base, _PER_W)], w_v)
    pltpu.sync_copy(m_hbm.at[pl.ds(base, _PER_W)], m_v)

    zero = jnp.zeros((_L,), jnp.float32)

    def body(i, carry):
        vec = w_v[pl.ds(i * _L, _L)]
        bits = m_v[pl.ds(i * _L, _L)]
        w_v[pl.ds(i * _L, _L)] = jnp.where(bits != 0, vec, zero)
        return carry

    lax.fori_loop(0, _PER_W // _L, body, 0)

    pltpu.sync_copy(w_v, out_hbm.at[pl.ds(base, _PER_W)])


def kernel(w, mask):
    w_flat = w.reshape(_N)
    m_i32 = mask.reshape(_N).astype(jnp.int32)
    out = _domain_mask_sc(w_flat, m_i32)
    return out.reshape(w.shape)


# speedup vs baseline: 1.2048x; 1.2048x over previous
import functools

import jax
import jax.numpy as jnp
from jax import lax
from jax.experimental import pallas as pl
from jax.experimental.pallas import tpu as pltpu
from jax.experimental.pallas import tpu_sc as plsc

_R, _C = 64, 32768
_N = _R * _C
_NC, _NS, _L = 2, 16, 16
_NW = _NC * _NS
_PER_W = _N // _NW
_WORDS_W = _PER_W // 4

_mesh = plsc.VectorSubcoreMesh(core_axis_name="c", subcore_axis_name="s")


@functools.partial(
    pl.kernel,
    out_type=jax.ShapeDtypeStruct((_N,), jnp.float32),
    mesh=_mesh,
    scratch_types=[
        pltpu.VMEM((_PER_W,), jnp.float32),
        pltpu.VMEM((_PER_W,), jnp.int32),
    ],
)
def _domain_mask_sc(w_hbm, m_hbm, out_hbm, w_v, m_v):
    wid = lax.axis_index("s") * _NC + lax.axis_index("c")
    base = wid * _PER_W

    pltpu.sync_copy(w_hbm.at[pl.ds(base, _PER_W)], w_v)
    pltpu.sync_copy(m_hbm.at[pl.ds(base, _PER_W)], m_v)

    zero = jnp.zeros((_L,), jnp.float32)

    @plsc.parallel_loop(0, _PER_W, step=_L, unroll=8)
    def _body(off):
        vec = w_v[pl.ds(off, _L)]
        bits = m_v[pl.ds(off, _L)]
        w_v[pl.ds(off, _L)] = jnp.where(bits != 0, vec, zero)

    pltpu.sync_copy(w_v, out_hbm.at[pl.ds(base, _PER_W)])


def kernel(w, mask):
    w_flat = w.reshape(_N)
    m_i32 = mask.reshape(_N).astype(jnp.int32)
    out = _domain_mask_sc(w_flat, m_i32)
    return out.reshape(w.shape)


# speedup vs baseline: 1.9963x; 1.6569x over previous
import functools

import jax
import jax.numpy as jnp
from jax import lax
from jax.experimental import pallas as pl
from jax.experimental.pallas import tpu as pltpu
from jax.experimental.pallas import tpu_sc as plsc

_R, _C = 64, 32768
_NC, _NS, _L = 2, 16, 16
_NW = _NC * _NS
_ROWS_W = _R // _NW
_CHUNK = 8192
_NCHUNK = _C // _CHUNK

_mesh = plsc.VectorSubcoreMesh(core_axis_name="c", subcore_axis_name="s")


@functools.partial(
    pl.kernel,
    out_type=jax.ShapeDtypeStruct((_R, _C), jnp.float32),
    mesh=_mesh,
    scratch_types=[
        pltpu.VMEM((_ROWS_W, _CHUNK), jnp.float32),
        pltpu.VMEM((_ROWS_W, _CHUNK), jnp.int32),
    ],
)
def _domain_mask_sc(w_hbm, m_hbm, out_hbm, w_v, m_v):
    wid = lax.axis_index("s") * _NC + lax.axis_index("c")
    r0 = wid * _ROWS_W

    zero = jnp.zeros((_L,), jnp.float32)

    for c in range(_NCHUNK):
        col = c * _CHUNK
        pltpu.sync_copy(w_hbm.at[pl.ds(r0, _ROWS_W), pl.ds(col, _CHUNK)], w_v)
        pltpu.sync_copy(m_hbm.at[pl.ds(r0, _ROWS_W), pl.ds(col, _CHUNK)], m_v)

        for r in range(_ROWS_W):
            @plsc.parallel_loop(0, _CHUNK, step=_L, unroll=8)
            def _body(off):
                vec = w_v[r, pl.ds(off, _L)]
                bits = m_v[r, pl.ds(off, _L)]
                w_v[r, pl.ds(off, _L)] = jnp.where(bits != 0, vec, zero)

        pltpu.sync_copy(w_v, out_hbm.at[pl.ds(r0, _ROWS_W), pl.ds(col, _CHUNK)])


def kernel(w, mask):
    return _domain_mask_sc(w, mask.astype(jnp.int32))
